# baseline (speedup 1.0000x reference)
import jax
import jax.numpy as jnp
from jax.experimental import pallas as pl
from jax.experimental.pallas import tpu as pltpu
from jax.experimental.pallas import tpu_sc as plsc

BATCH = 16384
HIDDEN = 64
CHUNK = 4096
NCHUNK = BATCH // CHUNK
NWORKERS = 32
BPW = BATCH // NWORKERS

N_ROWS = 100000
FMT_BC = 1024


def _fmt_body(t_ref, eye_ref, o_ref):
    o_ref[:, 0:HIDDEN] = jax.lax.dot_general(
        t_ref[...], eye_ref[...], (((0,), (0,)), ((), ())),
        precision=jax.lax.Precision.HIGHEST,
        preferred_element_type=jnp.float32)


def _fmt(table_t, eye):
    return pl.pallas_call(
        _fmt_body,
        grid=(pl.cdiv(N_ROWS, FMT_BC),),
        in_specs=[pl.BlockSpec((HIDDEN, FMT_BC), lambda i: (0, i)),
                  pl.BlockSpec((HIDDEN, HIDDEN), lambda i: (0, 0))],
        out_specs=pl.BlockSpec((FMT_BC, 2 * HIDDEN), lambda i: (i, 0)),
        out_shape=jax.ShapeDtypeStruct((N_ROWS, 2 * HIDDEN), jnp.float32),
    )(table_t, eye)


def _sc_gather_one(rows, idx):
    mesh = plsc.VectorSubcoreMesh(core_axis_name="core", subcore_axis_name="subcore")

    @pl.kernel(out_type=jax.ShapeDtypeStruct((BATCH, 2 * HIDDEN), jnp.float32),
               mesh=mesh,
               scratch_types=[
                   pltpu.VMEM((BPW,), jnp.int32),
                   pltpu.VMEM((BPW, 2 * HIDDEN), jnp.float32),
                   pltpu.SemaphoreType.DMA,
               ])
    def gather_kernel(t_hbm, i_hbm, o_hbm, i_v, rows_v, sem):
        wid = jax.lax.axis_index("subcore") * 2 + jax.lax.axis_index("core")
        base = wid * BPW
        pltpu.sync_copy(i_hbm.at[0, pl.ds(base, BPW)], i_v)
        pltpu.async_copy(t_hbm.at[i_v], rows_v, sem).wait()
        pltpu.sync_copy(rows_v, o_hbm.at[pl.ds(base, BPW)])

    return gather_kernel(rows, idx)


def _dot(a, b):
    return jax.lax.dot_general(
        a, b, (((1,), (0,)), ((), ())), preferred_element_type=jnp.float32)


def _mlp_body(p_ref, n_ref, w1p_ref, w1n_ref, b1_ref, gamma_ref,
              beta_ref, w2_ref, b2_ref, w3_ref, b3_ref, out_ref,
              h1_ref, stat_ref):
    phase = pl.program_id(0)
    chunk = pl.program_id(1)

    @pl.when(phase == 0)
    def _pass1():
        @pl.when(chunk == 0)
        def _init():
            stat_ref[...] = jnp.zeros_like(stat_ref)

        p = p_ref[:, 0:HIDDEN]
        n = n_ref[:, 0:HIDDEN]
        p_norm = jnp.sqrt(jnp.sum(p * p, axis=-1, keepdims=True))
        n_norm = jnp.sqrt(jnp.sum(n * n, axis=-1, keepdims=True))
        p = p / jnp.maximum(p_norm, 1e-12)
        n = n / jnp.maximum(n_norm, 1e-12)
        h = _dot(p, w1p_ref[...]) + _dot(n, w1n_ref[...]) + b1_ref[...]
        h = jnp.maximum(h, 0.0)
        h1_ref[pl.ds(chunk * CHUNK, CHUNK), :] = h
        stat_ref[0:1, :] += jnp.sum(h, axis=0, keepdims=True)
        stat_ref[1:2, :] += jnp.sum(h * h, axis=0, keepdims=True)

    @pl.when(phase == 1)
    def _pass2():
        inv_n = 1.0 / BATCH
        mean = stat_ref[0:1, :] * inv_n
        var = stat_ref[1:2, :] * inv_n - mean * mean
        h = h1_ref[pl.ds(chunk * CHUNK, CHUNK), :]
        h = (h - mean) / jnp.sqrt(var + 1e-5) * gamma_ref[...] + beta_ref[...]
        h = jnp.maximum(_dot(h, w2_ref[...]) + b2_ref[...], 0.0)
        out_ref[...] = _dot(h, w3_ref[...]) + b3_ref[...]


def _mlp(p_rows, n_rows, W1, b1, gamma, beta, W2, b2, W3, b3, *,
         interpret=False):
    n_obs = W3.shape[0]

    def chunk_map(ph, c):
        return (jnp.where(ph == 0, c, 0), 0)

    def bcast_map(ph, c):
        return (0, 0)

    return pl.pallas_call(
        _mlp_body,
        grid=(2, NCHUNK),
        in_specs=[
            pl.BlockSpec((CHUNK, 2 * HIDDEN), chunk_map),
            pl.BlockSpec((CHUNK, 2 * HIDDEN), chunk_map),
            pl.BlockSpec((HIDDEN, HIDDEN), bcast_map),
            pl.BlockSpec((HIDDEN, HIDDEN), bcast_map),
            pl.BlockSpec((1, HIDDEN), bcast_map),
            pl.BlockSpec((1, HIDDEN), bcast_map),
            pl.BlockSpec((1, HIDDEN), bcast_map),
            pl.BlockSpec((HIDDEN, HIDDEN), bcast_map),
            pl.BlockSpec((1, HIDDEN), bcast_map),
            pl.BlockSpec((HIDDEN, n_obs), bcast_map),
            pl.BlockSpec((1, n_obs), bcast_map),
        ],
        out_specs=pl.BlockSpec((CHUNK, n_obs),
                               lambda ph, c: (jnp.where(ph == 1, c, 0), 0)),
        out_shape=jax.ShapeDtypeStruct((BATCH, n_obs), jnp.float32),
        scratch_shapes=[
            pltpu.VMEM((BATCH, HIDDEN), jnp.float32),
            pltpu.VMEM((2, HIDDEN), jnp.float32),
        ],
        interpret=interpret,
    )(
        p_rows,
        n_rows,
        W1[:, :HIDDEN].T,
        W1[:, HIDDEN:].T,
        b1.reshape(1, -1),
        gamma.reshape(1, -1),
        beta.reshape(1, -1),
        W2.T,
        b2.reshape(1, -1),
        W3.T,
        b3.reshape(1, -1),
    )


def kernel(x, pos_proton, pos_neutron, W1, b1, gamma, beta, W2, b2, W3, b3):
    idx_p = x[:, 0].reshape(1, BATCH)
    idx_n = x[:, 1].reshape(1, BATCH)
    eye = jnp.eye(HIDDEN, dtype=jnp.float32)
    fp = _fmt(pos_proton.T, eye)
    p_rows = _sc_gather_one(fp, idx_p)
    fn = _fmt(pos_neutron.T, eye)
    n_rows = _sc_gather_one(fn, idx_n)
    return _mlp(p_rows, n_rows, W1, b1, gamma, beta, W2, b2, W3, b3)


# speedup vs baseline: 1.0004x; 1.0004x over previous
import jax
import jax.numpy as jnp
from jax.experimental import pallas as pl
from jax.experimental.pallas import tpu as pltpu
from jax.experimental.pallas import tpu_sc as plsc

BATCH = 16384
HIDDEN = 64
CHUNK = 4096
NCHUNK = BATCH // CHUNK
NWORKERS = 32
BPW = BATCH // NWORKERS

N_ROWS = 100000
FMT_BC = 1024


def _fmt_body(t_ref, eye_ref, o_ref):
    o_ref[:, 0:HIDDEN] = jax.lax.dot_general(
        t_ref[...], eye_ref[...], (((0,), (0,)), ((), ())),
        precision=jax.lax.Precision.HIGHEST,
        preferred_element_type=jnp.float32)


def _fmt(table_t, eye):
    return pl.pallas_call(
        _fmt_body,
        grid=(pl.cdiv(N_ROWS, FMT_BC),),
        in_specs=[pl.BlockSpec((HIDDEN, FMT_BC), lambda i: (0, i)),
                  pl.BlockSpec((HIDDEN, HIDDEN), lambda i: (0, 0))],
        out_specs=pl.BlockSpec((FMT_BC, 2 * HIDDEN), lambda i: (i, 0)),
        out_shape=jax.ShapeDtypeStruct((N_ROWS, 2 * HIDDEN), jnp.float32),
        compiler_params=pltpu.CompilerParams(fuse_transposed_lhs_in_matmul=True),
    )(table_t, eye)


def _sc_gather_one(rows, idx):
    mesh = plsc.VectorSubcoreMesh(core_axis_name="core", subcore_axis_name="subcore")

    @pl.kernel(out_type=jax.ShapeDtypeStruct((BATCH, 2 * HIDDEN), jnp.float32),
               mesh=mesh,
               scratch_types=[
                   pltpu.VMEM((BPW,), jnp.int32),
                   pltpu.VMEM((BPW, 2 * HIDDEN), jnp.float32),
                   pltpu.SemaphoreType.DMA,
               ])
    def gather_kernel(t_hbm, i_hbm, o_hbm, i_v, rows_v, sem):
        wid = jax.lax.axis_index("subcore") * 2 + jax.lax.axis_index("core")
        base = wid * BPW
        pltpu.sync_copy(i_hbm.at[0, pl.ds(base, BPW)], i_v)
        pltpu.async_copy(t_hbm.at[i_v], rows_v, sem).wait()
        pltpu.sync_copy(rows_v, o_hbm.at[pl.ds(base, BPW)])

    return gather_kernel(rows, idx)


def _dot(a, b):
    return jax.lax.dot_general(
        a, b, (((1,), (0,)), ((), ())), preferred_element_type=jnp.float32)


def _mlp_body(p_ref, n_ref, w1p_ref, w1n_ref, b1_ref, gamma_ref,
              beta_ref, w2_ref, b2_ref, w3_ref, b3_ref, out_ref,
              h1_ref, stat_ref):
    phase = pl.program_id(0)
    chunk = pl.program_id(1)

    @pl.when(phase == 0)
    def _pass1():
        @pl.when(chunk == 0)
        def _init():
            stat_ref[...] = jnp.zeros_like(stat_ref)

        p = p_ref[:, 0:HIDDEN]
        n = n_ref[:, 0:HIDDEN]
        p_norm = jnp.sqrt(jnp.sum(p * p, axis=-1, keepdims=True))
        n_norm = jnp.sqrt(jnp.sum(n * n, axis=-1, keepdims=True))
        p = p / jnp.maximum(p_norm, 1e-12)
        n = n / jnp.maximum(n_norm, 1e-12)
        h = _dot(p, w1p_ref[...]) + _dot(n, w1n_ref[...]) + b1_ref[...]
        h = jnp.maximum(h, 0.0)
        h1_ref[pl.ds(chunk * CHUNK, CHUNK), :] = h
        stat_ref[0:1, :] += jnp.sum(h, axis=0, keepdims=True)
        stat_ref[1:2, :] += jnp.sum(h * h, axis=0, keepdims=True)

    @pl.when(phase == 1)
    def _pass2():
        inv_n = 1.0 / BATCH
        mean = stat_ref[0:1, :] * inv_n
        var = stat_ref[1:2, :] * inv_n - mean * mean
        h = h1_ref[pl.ds(chunk * CHUNK, CHUNK), :]
        h = (h - mean) / jnp.sqrt(var + 1e-5) * gamma_ref[...] + beta_ref[...]
        h = jnp.maximum(_dot(h, w2_ref[...]) + b2_ref[...], 0.0)
        out_ref[...] = _dot(h, w3_ref[...]) + b3_ref[...]


def _mlp(p_rows, n_rows, W1, b1, gamma, beta, W2, b2, W3, b3, *,
         interpret=False):
    n_obs = W3.shape[0]

    def chunk_map(ph, c):
        return (jnp.where(ph == 0, c, 0), 0)

    def bcast_map(ph, c):
        return (0, 0)

    return pl.pallas_call(
        _mlp_body,
        grid=(2, NCHUNK),
        in_specs=[
            pl.BlockSpec((CHUNK, 2 * HIDDEN), chunk_map),
            pl.BlockSpec((CHUNK, 2 * HIDDEN), chunk_map),
            pl.BlockSpec((HIDDEN, HIDDEN), bcast_map),
            pl.BlockSpec((HIDDEN, HIDDEN), bcast_map),
            pl.BlockSpec((1, HIDDEN), bcast_map),
            pl.BlockSpec((1, HIDDEN), bcast_map),
            pl.BlockSpec((1, HIDDEN), bcast_map),
            pl.BlockSpec((HIDDEN, HIDDEN), bcast_map),
            pl.BlockSpec((1, HIDDEN), bcast_map),
            pl.BlockSpec((HIDDEN, n_obs), bcast_map),
            pl.BlockSpec((1, n_obs), bcast_map),
        ],
        out_specs=pl.BlockSpec((CHUNK, n_obs),
                               lambda ph, c: (jnp.where(ph == 1, c, 0), 0)),
        out_shape=jax.ShapeDtypeStruct((BATCH, n_obs), jnp.float32),
        scratch_shapes=[
            pltpu.VMEM((BATCH, HIDDEN), jnp.float32),
            pltpu.VMEM((2, HIDDEN), jnp.float32),
        ],
        interpret=interpret,
    )(
        p_rows,
        n_rows,
        W1[:, :HIDDEN].T,
        W1[:, HIDDEN:].T,
        b1.reshape(1, -1),
        gamma.reshape(1, -1),
        beta.reshape(1, -1),
        W2.T,
        b2.reshape(1, -1),
        W3.T,
        b3.reshape(1, -1),
    )


def kernel(x, pos_proton, pos_neutron, W1, b1, gamma, beta, W2, b2, W3, b3):
    idx_p = x[:, 0].reshape(1, BATCH)
    idx_n = x[:, 1].reshape(1, BATCH)
    eye = jnp.eye(HIDDEN, dtype=jnp.float32)
    fp = _fmt(pos_proton.T, eye)
    p_rows = _sc_gather_one(fp, idx_p)
    fn = _fmt(pos_neutron.T, eye)
    n_rows = _sc_gather_one(fn, idx_n)
    return _mlp(p_rows, n_rows, W1, b1, gamma, beta, W2, b2, W3, b3)


# speedup vs baseline: 1.1706x; 1.1701x over previous
import jax
import jax.numpy as jnp
from jax.experimental import pallas as pl
from jax.experimental.pallas import tpu as pltpu
from jax.experimental.pallas import tpu_sc as plsc

BATCH = 16384
HIDDEN = 64
CHUNK = 4096
NCHUNK = BATCH // CHUNK
NWORKERS = 32
BPW = BATCH // NWORKERS

N_ROWS = 100000
FMT_BC = 1024


def _fmt_body(t_ref, o_ref):
    o_ref[:, 0:HIDDEN] = t_ref[...].T


def _fmt(table_t):
    return pl.pallas_call(
        _fmt_body,
        grid=(pl.cdiv(N_ROWS, FMT_BC),),
        in_specs=[pl.BlockSpec((HIDDEN, FMT_BC), lambda i: (0, i))],
        out_specs=pl.BlockSpec((FMT_BC, 2 * HIDDEN), lambda i: (i, 0)),
        out_shape=jax.ShapeDtypeStruct((N_ROWS, 2 * HIDDEN), jnp.float32),
        compiler_params=pltpu.CompilerParams(
            dimension_semantics=("parallel",)),
    )(table_t)


def _sc_gather_one(rows, idx):
    mesh = plsc.VectorSubcoreMesh(core_axis_name="core", subcore_axis_name="subcore")

    @pl.kernel(out_type=jax.ShapeDtypeStruct((BATCH, 2 * HIDDEN), jnp.float32),
               mesh=mesh,
               scratch_types=[
                   pltpu.VMEM((BPW,), jnp.int32),
                   pltpu.VMEM((BPW, 2 * HIDDEN), jnp.float32),
                   pltpu.SemaphoreType.DMA,
               ])
    def gather_kernel(t_hbm, i_hbm, o_hbm, i_v, rows_v, sem):
        wid = jax.lax.axis_index("subcore") * 2 + jax.lax.axis_index("core")
        base = wid * BPW
        pltpu.sync_copy(i_hbm.at[0, pl.ds(base, BPW)], i_v)
        pltpu.async_copy(t_hbm.at[i_v], rows_v, sem).wait()
        pltpu.sync_copy(rows_v, o_hbm.at[pl.ds(base, BPW)])

    return gather_kernel(rows, idx)


def _dot(a, b):
    return jax.lax.dot_general(
        a, b, (((1,), (0,)), ((), ())), preferred_element_type=jnp.float32)


def _mlp_body(p_ref, n_ref, w1p_ref, w1n_ref, b1_ref, gamma_ref,
              beta_ref, w2_ref, b2_ref, w3_ref, b3_ref, out_ref,
              h1_ref, stat_ref):
    phase = pl.program_id(0)
    chunk = pl.program_id(1)

    @pl.when(phase == 0)
    def _pass1():
        @pl.when(chunk == 0)
        def _init():
            stat_ref[...] = jnp.zeros_like(stat_ref)

        p = p_ref[:, 0:HIDDEN]
        n = n_ref[:, 0:HIDDEN]
        p_norm = jnp.sqrt(jnp.sum(p * p, axis=-1, keepdims=True))
        n_norm = jnp.sqrt(jnp.sum(n * n, axis=-1, keepdims=True))
        p = p / jnp.maximum(p_norm, 1e-12)
        n = n / jnp.maximum(n_norm, 1e-12)
        h = _dot(p, w1p_ref[...]) + _dot(n, w1n_ref[...]) + b1_ref[...]
        h = jnp.maximum(h, 0.0)
        h1_ref[pl.ds(chunk * CHUNK, CHUNK), :] = h
        stat_ref[0:1, :] += jnp.sum(h, axis=0, keepdims=True)
        stat_ref[1:2, :] += jnp.sum(h * h, axis=0, keepdims=True)

    @pl.when(phase == 1)
    def _pass2():
        inv_n = 1.0 / BATCH
        mean = stat_ref[0:1, :] * inv_n
        var = stat_ref[1:2, :] * inv_n - mean * mean
        h = h1_ref[pl.ds(chunk * CHUNK, CHUNK), :]
        h = (h - mean) / jnp.sqrt(var + 1e-5) * gamma_ref[...] + beta_ref[...]
        h = jnp.maximum(_dot(h, w2_ref[...]) + b2_ref[...], 0.0)
        out_ref[...] = _dot(h, w3_ref[...]) + b3_ref[...]


def _mlp(p_rows, n_rows, W1, b1, gamma, beta, W2, b2, W3, b3, *,
         interpret=False):
    n_obs = W3.shape[0]

    def chunk_map(ph, c):
        return (jnp.where(ph == 0, c, 0), 0)

    def bcast_map(ph, c):
        return (0, 0)

    return pl.pallas_call(
        _mlp_body,
        grid=(2, NCHUNK),
        in_specs=[
            pl.BlockSpec((CHUNK, 2 * HIDDEN), chunk_map),
            pl.BlockSpec((CHUNK, 2 * HIDDEN), chunk_map),
            pl.BlockSpec((HIDDEN, HIDDEN), bcast_map),
            pl.BlockSpec((HIDDEN, HIDDEN), bcast_map),
            pl.BlockSpec((1, HIDDEN), bcast_map),
            pl.BlockSpec((1, HIDDEN), bcast_map),
            pl.BlockSpec((1, HIDDEN), bcast_map),
            pl.BlockSpec((HIDDEN, HIDDEN), bcast_map),
            pl.BlockSpec((1, HIDDEN), bcast_map),
            pl.BlockSpec((HIDDEN, n_obs), bcast_map),
            pl.BlockSpec((1, n_obs), bcast_map),
        ],
        out_specs=pl.BlockSpec((CHUNK, n_obs),
                               lambda ph, c: (jnp.where(ph == 1, c, 0), 0)),
        out_shape=jax.ShapeDtypeStruct((BATCH, n_obs), jnp.float32),
        scratch_shapes=[
            pltpu.VMEM((BATCH, HIDDEN), jnp.float32),
            pltpu.VMEM((2, HIDDEN), jnp.float32),
        ],
        interpret=interpret,
    )(
        p_rows,
        n_rows,
        W1[:, :HIDDEN].T,
        W1[:, HIDDEN:].T,
        b1.reshape(1, -1),
        gamma.reshape(1, -1),
        beta.reshape(1, -1),
        W2.T,
        b2.reshape(1, -1),
        W3.T,
        b3.reshape(1, -1),
    )


def kernel(x, pos_proton, pos_neutron, W1, b1, gamma, beta, W2, b2, W3, b3):
    idx_p = x[:, 0].reshape(1, BATCH)
    idx_n = x[:, 1].reshape(1, BATCH)
    fp = _fmt(pos_proton.T)
    p_rows = _sc_gather_one(fp, idx_p)
    fn = _fmt(pos_neutron.T)
    n_rows = _sc_gather_one(fn, idx_n)
    return _mlp(p_rows, n_rows, W1, b1, gamma, beta, W2, b2, W3, b3)


# speedup vs baseline: 1.3190x; 1.1268x over previous
import jax
import jax.numpy as jnp
from jax.experimental import pallas as pl
from jax.experimental.pallas import tpu as pltpu
from jax.experimental.pallas import tpu_sc as plsc

BATCH = 16384
HIDDEN = 64
CHUNK = 4096
NCHUNK = BATCH // CHUNK
NWORKERS = 32
BPW = BATCH // NWORKERS

N_ROWS = 100000
FMT_BC = 1024


def _sc_gather_one(table, xt, row):
    mesh = plsc.VectorSubcoreMesh(core_axis_name="core", subcore_axis_name="subcore")

    @pl.kernel(out_type=jax.ShapeDtypeStruct((BATCH, HIDDEN), jnp.float32),
               mesh=mesh,
               compiler_params=pltpu.CompilerParams(use_tc_tiling_on_sc=False),
               scratch_types=[
                   pltpu.VMEM((BPW,), jnp.int32),
                   pltpu.VMEM((BPW, HIDDEN), jnp.float32),
                   pltpu.SemaphoreType.DMA,
               ])
    def gather_kernel(t_hbm, i_hbm, o_hbm, i_v, rows_v, sem):
        wid = jax.lax.axis_index("subcore") * 2 + jax.lax.axis_index("core")
        base = wid * BPW
        pltpu.sync_copy(i_hbm.at[row, pl.ds(base, BPW)], i_v)
        pltpu.async_copy(t_hbm.at[i_v], rows_v, sem).wait()
        pltpu.sync_copy(rows_v, o_hbm.at[pl.ds(base, BPW)])

    return gather_kernel(table, xt)


def _dot(a, b):
    return jax.lax.dot_general(
        a, b, (((1,), (0,)), ((), ())), preferred_element_type=jnp.float32)


def _mlp_body(p_ref, n_ref, w1p_ref, w1n_ref, b1_ref, gamma_ref,
              beta_ref, w2_ref, b2_ref, w3_ref, b3_ref, out_ref,
              h1_ref, stat_ref):
    phase = pl.program_id(0)
    chunk = pl.program_id(1)

    @pl.when(phase == 0)
    def _pass1():
        @pl.when(chunk == 0)
        def _init():
            stat_ref[...] = jnp.zeros_like(stat_ref)

        p = p_ref[...]
        n = n_ref[...]
        p_norm = jnp.sqrt(jnp.sum(p * p, axis=-1, keepdims=True))
        n_norm = jnp.sqrt(jnp.sum(n * n, axis=-1, keepdims=True))
        p = p / jnp.maximum(p_norm, 1e-12)
        n = n / jnp.maximum(n_norm, 1e-12)
        h = _dot(p, w1p_ref[...]) + _dot(n, w1n_ref[...]) + b1_ref[...]
        h = jnp.maximum(h, 0.0)
        h1_ref[pl.ds(chunk * CHUNK, CHUNK), :] = h
        stat_ref[0:1, :] += jnp.sum(h, axis=0, keepdims=True)
        stat_ref[1:2, :] += jnp.sum(h * h, axis=0, keepdims=True)

    @pl.when(phase == 1)
    def _pass2():
        inv_n = 1.0 / BATCH
        mean = stat_ref[0:1, :] * inv_n
        var = stat_ref[1:2, :] * inv_n - mean * mean
        h = h1_ref[pl.ds(chunk * CHUNK, CHUNK), :]
        h = (h - mean) / jnp.sqrt(var + 1e-5) * gamma_ref[...] + beta_ref[...]
        h = jnp.maximum(_dot(h, w2_ref[...]) + b2_ref[...], 0.0)
        out_ref[...] = _dot(h, w3_ref[...]) + b3_ref[...]


def _mlp(p_rows, n_rows, W1, b1, gamma, beta, W2, b2, W3, b3, *,
         interpret=False):
    n_obs = W3.shape[0]

    def chunk_map(ph, c):
        return (jnp.where(ph == 0, c, 0), 0)

    def bcast_map(ph, c):
        return (0, 0)

    return pl.pallas_call(
        _mlp_body,
        grid=(2, NCHUNK),
        in_specs=[
            pl.BlockSpec((CHUNK, HIDDEN), chunk_map),
            pl.BlockSpec((CHUNK, HIDDEN), chunk_map),
            pl.BlockSpec((HIDDEN, HIDDEN), bcast_map),
            pl.BlockSpec((HIDDEN, HIDDEN), bcast_map),
            pl.BlockSpec((1, HIDDEN), bcast_map),
            pl.BlockSpec((1, HIDDEN), bcast_map),
            pl.BlockSpec((1, HIDDEN), bcast_map),
            pl.BlockSpec((HIDDEN, HIDDEN), bcast_map),
            pl.BlockSpec((1, HIDDEN), bcast_map),
            pl.BlockSpec((HIDDEN, n_obs), bcast_map),
            pl.BlockSpec((1, n_obs), bcast_map),
        ],
        out_specs=pl.BlockSpec((CHUNK, n_obs),
                               lambda ph, c: (jnp.where(ph == 1, c, 0), 0)),
        out_shape=jax.ShapeDtypeStruct((BATCH, n_obs), jnp.float32),
        scratch_shapes=[
            pltpu.VMEM((BATCH, HIDDEN), jnp.float32),
            pltpu.VMEM((2, HIDDEN), jnp.float32),
        ],
        interpret=interpret,
    )(
        p_rows,
        n_rows,
        W1[:, :HIDDEN].T,
        W1[:, HIDDEN:].T,
        b1.reshape(1, -1),
        gamma.reshape(1, -1),
        beta.reshape(1, -1),
        W2.T,
        b2.reshape(1, -1),
        W3.T,
        b3.reshape(1, -1),
    )


def kernel(x, pos_proton, pos_neutron, W1, b1, gamma, beta, W2, b2, W3, b3):
    xt = x.T
    p_rows = _sc_gather_one(pos_proton, xt, 0)
    n_rows = _sc_gather_one(pos_neutron, xt, 1)
    return _mlp(p_rows, n_rows, W1, b1, gamma, beta, W2, b2, W3, b3)
